# baseline (speedup 1.0000x reference)
import functools

import jax
import jax.numpy as jnp
from jax import lax
from jax.experimental import pallas as pl
from jax.experimental.pallas import tpu as pltpu
from jax.experimental.pallas import tpu_sc as plsc

MUL = 8
FC_IN = 16
FC_HID = 64
WNUM = MUL * MUL * MUL
ACT_CST = 1.679
NUM_NEIGHBORS = 16.0

NC, NS = 2, 16
NW = NC * NS
CHUNK = 128

_MESH = dict(core_axis_name="c", subcore_axis_name="s")


@functools.lru_cache(maxsize=None)
def _gather_fn(E_pad, N):
    mesh = plsc.VectorSubcoreMesh(**_MESH)
    epw = E_pad // NW

    @functools.partial(
        pl.kernel,
        out_type=(
            jax.ShapeDtypeStruct((E_pad, MUL), jnp.float32),
            jax.ShapeDtypeStruct((E_pad, MUL), jnp.float32),
        ),
        mesh=mesh,
        scratch_types=[
            pltpu.VMEM((epw,), jnp.int32),
            pltpu.VMEM((epw,), jnp.int32),
            pltpu.VMEM((epw, MUL), jnp.float32),
            pltpu.VMEM((epw, MUL), jnp.float32),
            pltpu.SemaphoreType.DMA,
        ],
        compiler_params=pltpu.CompilerParams(use_tc_tiling_on_sc=False),
    )
    def k(src_hbm, dst_hbm, emb_hbm, src_out, dst_out,
          sidx, didx, srows, drows, sem):
        wid = lax.axis_index("s") * NC + lax.axis_index("c")
        base = wid * epw
        pltpu.sync_copy(src_hbm.at[pl.ds(base, epw)], sidx)
        pltpu.sync_copy(dst_hbm.at[pl.ds(base, epw)], didx)
        c1 = pltpu.async_copy(emb_hbm.at[sidx], srows, sem)
        c2 = pltpu.async_copy(emb_hbm.at[didx], drows, sem)
        c1.wait()
        c2.wait()
        pltpu.sync_copy(srows, src_out.at[pl.ds(base, epw)])
        pltpu.sync_copy(drows, dst_out.at[pl.ds(base, epw)])

    return k


@functools.lru_cache(maxsize=None)
def _scatter_fn(N, cpw):
    mesh = plsc.VectorSubcoreMesh(**_MESH)
    epw = cpw * CHUNK

    @functools.partial(
        pl.kernel,
        out_type=jax.ShapeDtypeStruct((NC, N, MUL), jnp.float32),
        mesh=mesh,
        scratch_types=[
            pltpu.VMEM((cpw, CHUNK), jnp.int32),
            pltpu.VMEM((epw, MUL), jnp.float32),
            pltpu.VMEM_SHARED((N, MUL), jnp.float32),
            pltpu.SemaphoreType.DMA,
        ],
        compiler_params=pltpu.CompilerParams(use_tc_tiling_on_sc=False),
    )
    def k(didx_hbm, msg_hbm, zero_hbm, out_hbm, idxv, msgv, acc, sem):
        cid = lax.axis_index("c")
        sid = lax.axis_index("s")
        wid = sid * NC + cid

        @pl.when(sid == 0)
        def _():
            pltpu.sync_copy(zero_hbm, acc)

        pltpu.sync_copy(didx_hbm.at[wid], idxv)
        pltpu.sync_copy(msg_hbm.at[pl.ds(wid * epw, epw)], msgv)
        plsc.subcore_barrier()

        def body(j, carry):
            pltpu.sync_copy(msgv.at[pl.ds(j * CHUNK, CHUNK)],
                            acc.at[idxv.at[j]], add=True)
            return carry

        lax.fori_loop(0, cpw, body, 0)
        plsc.subcore_barrier()

        @pl.when(sid == 0)
        def _():
            pltpu.sync_copy(acc, out_hbm.at[cid])

    return k


def _dense_body(et_ref, x_ref, y_ref, w1_ref, w2_ref,
                lm_ref, pm_ref, bsel_ref, qx_ref, qy_ref, qo_ref, selk_ref,
                out_ref):
    et = et_ref[...]
    h = jax.nn.silu(
        jnp.dot(et, w1_ref[...], preferred_element_type=jnp.float32) * 0.25
    ) * ACT_CST
    w = jnp.dot(h, w2_ref[...], preferred_element_type=jnp.float32)

    bf = jnp.bfloat16
    lm = lm_ref[...]
    bsel = bsel_ref[...]
    tx = jnp.dot(lm, x_ref[...].astype(bf),
                 preferred_element_type=jnp.float32).astype(bf) * bsel
    ty = jnp.dot(lm, y_ref[...].astype(bf),
                 preferred_element_type=jnp.float32).astype(bf) * bsel
    x64 = jnp.dot(tx, qx_ref[...], preferred_element_type=jnp.float32)
    y64 = jnp.dot(ty, qy_ref[...], preferred_element_type=jnp.float32)
    o = (x64 * y64).astype(bf)
    oe = jnp.dot(o, qo_ref[...], preferred_element_type=jnp.float32)
    prod = w * oe
    m128 = jnp.dot(prod.astype(bf), selk_ref[...],
                   preferred_element_type=jnp.float32)
    out_ref[...] = jnp.dot(pm_ref[...], (m128 * bsel).astype(bf),
                           preferred_element_type=jnp.float32)


def _dense_consts(B):
    scale = 1.0 / (8.0 * MUL * float(NUM_NEIGHBORS) ** 0.5)
    e = jnp.arange(B, dtype=jnp.int32)
    r = jnp.arange(B // 16, dtype=jnp.int32)
    l = jnp.arange(128, dtype=jnp.int32)
    m = jnp.arange(WNUM, dtype=jnp.int32)
    bf = jnp.bfloat16
    ij = jnp.arange(MUL * MUL, dtype=jnp.int32)
    lm = (e[:, None] // 16 == r[None, :]).astype(bf)
    pm = (r[:, None] == e[None, :] // 16).astype(bf)
    bsel = (l[None, :] // MUL == e[:, None] % 16).astype(bf)
    qx = (l[:, None] % MUL == ij[None, :] // MUL).astype(bf)
    qy = (l[:, None] % MUL == ij[None, :] % MUL).astype(bf)
    qo = (ij[:, None] == m[None, :] // MUL).astype(bf)
    selk = (m[:, None] % MUL == l[None, :] % MUL).astype(bf) * bf(scale)
    return lm, pm, bsel, qx, qy, qo, selk


def _dense_fn(E_pad, B):
    grid = E_pad // B
    R = B * MUL // 128
    return pl.pallas_call(
        _dense_body,
        grid=(grid,),
        in_specs=[
            pl.BlockSpec((B, FC_IN), lambda i: (i, 0)),
            pl.BlockSpec((R, 128), lambda i: (i, 0)),
            pl.BlockSpec((R, 128), lambda i: (i, 0)),
            pl.BlockSpec((FC_IN, FC_HID), lambda i: (0, 0)),
            pl.BlockSpec((FC_HID, WNUM), lambda i: (0, 0)),
            pl.BlockSpec((B, R), lambda i: (0, 0)),
            pl.BlockSpec((R, B), lambda i: (0, 0)),
            pl.BlockSpec((B, 128), lambda i: (0, 0)),
            pl.BlockSpec((128, MUL * MUL), lambda i: (0, 0)),
            pl.BlockSpec((128, MUL * MUL), lambda i: (0, 0)),
            pl.BlockSpec((MUL * MUL, WNUM), lambda i: (0, 0)),
            pl.BlockSpec((WNUM, 128), lambda i: (0, 0)),
        ],
        out_specs=pl.BlockSpec((R, 128), lambda i: (i, 0)),
        out_shape=jax.ShapeDtypeStruct((E_pad * MUL // 128, 128), jnp.float32),
        compiler_params=pltpu.CompilerParams(
            dimension_semantics=("parallel",)),
    )


def _combine_body(p_ref, o_ref):
    o_ref[...] = p_ref[0] + p_ref[1]


def _combine_fn(N):
    return pl.pallas_call(
        _combine_body,
        out_shape=jax.ShapeDtypeStruct((N, MUL), jnp.float32),
    )


def kernel(edge_src, edge_dst, node_emb, edge_type, W1, W2):
    E = edge_src.shape[0]
    N = node_emb.shape[1]
    quantum = NW * CHUNK
    E_pad = ((E + quantum - 1) // quantum) * quantum
    cpw = E_pad // quantum
    pad = E_pad - E

    pad_idx = jnp.arange(pad, dtype=jnp.int32) % N
    src_p = jnp.concatenate([edge_src, pad_idx])
    dst_p = jnp.concatenate([edge_dst, pad_idx])
    et_p = jnp.concatenate(
        [edge_type, jnp.zeros((pad, edge_type.shape[1]), edge_type.dtype)])
    emb = node_emb[0]

    src_emb, dst_emb = _gather_fn(E_pad, N)(src_p, dst_p, emb)
    xp = src_emb.reshape(E_pad * MUL // 128, 128)
    yp = dst_emb.reshape(E_pad * MUL // 128, 128)

    msgs = _dense_fn(E_pad, 2048)(et_p, xp, yp, W1, W2, *_dense_consts(2048))

    partials = _scatter_fn(N, cpw)(
        dst_p.reshape(NW, cpw, CHUNK), msgs.reshape(E_pad, MUL),
        jnp.zeros((N, MUL), jnp.float32))

    out = _combine_fn(N)(partials)
    return out.reshape(node_emb.shape)


# speedup vs baseline: 1.0252x; 1.0252x over previous
import functools

import jax
import jax.numpy as jnp
from jax import lax
from jax.experimental import pallas as pl
from jax.experimental.pallas import tpu as pltpu
from jax.experimental.pallas import tpu_sc as plsc

MUL = 8
FC_IN = 16
FC_HID = 64
WNUM = MUL * MUL * MUL
ACT_CST = 1.679
NUM_NEIGHBORS = 16.0

NC, NS = 2, 16
NW = NC * NS
CHUNK = 128

_MESH = dict(core_axis_name="c", subcore_axis_name="s")


@functools.lru_cache(maxsize=None)
def _gather_fn(E_pad, N):
    mesh = plsc.VectorSubcoreMesh(**_MESH)
    epw = E_pad // NW

    @functools.partial(
        pl.kernel,
        out_type=(
            jax.ShapeDtypeStruct((E_pad, MUL), jnp.float32),
            jax.ShapeDtypeStruct((E_pad, MUL), jnp.float32),
        ),
        mesh=mesh,
        scratch_types=[
            pltpu.VMEM((epw,), jnp.int32),
            pltpu.VMEM((epw,), jnp.int32),
            pltpu.VMEM((epw, MUL), jnp.float32),
            pltpu.VMEM((epw, MUL), jnp.float32),
            pltpu.SemaphoreType.DMA,
        ],
        compiler_params=pltpu.CompilerParams(use_tc_tiling_on_sc=False),
    )
    def k(src_hbm, dst_hbm, emb_hbm, src_out, dst_out,
          sidx, didx, srows, drows, sem):
        wid = lax.axis_index("s") * NC + lax.axis_index("c")
        base = wid * epw
        pltpu.sync_copy(src_hbm.at[pl.ds(base, epw)], sidx)
        pltpu.sync_copy(dst_hbm.at[pl.ds(base, epw)], didx)
        c1 = pltpu.async_copy(emb_hbm.at[sidx], srows, sem)
        c2 = pltpu.async_copy(emb_hbm.at[didx], drows, sem)
        c1.wait()
        c2.wait()
        pltpu.sync_copy(srows, src_out.at[pl.ds(base, epw)])
        pltpu.sync_copy(drows, dst_out.at[pl.ds(base, epw)])

    return k


@functools.lru_cache(maxsize=None)
def _scatter_fn(N, cpw):
    mesh = plsc.VectorSubcoreMesh(**_MESH)
    epw = cpw * CHUNK

    @functools.partial(
        pl.kernel,
        out_type=jax.ShapeDtypeStruct((NC, N, MUL), jnp.float32),
        mesh=mesh,
        scratch_types=[
            pltpu.VMEM((cpw, CHUNK), jnp.int32),
            pltpu.VMEM((epw, MUL), jnp.float32),
            pltpu.VMEM_SHARED((N, MUL), jnp.float32),
            pltpu.SemaphoreType.DMA,
        ],
        compiler_params=pltpu.CompilerParams(use_tc_tiling_on_sc=False),
    )
    def k(didx_hbm, msg_hbm, zero_hbm, out_hbm, idxv, msgv, acc, sem):
        cid = lax.axis_index("c")
        sid = lax.axis_index("s")
        wid = sid * NC + cid

        @pl.when(sid == 0)
        def _():
            pltpu.sync_copy(zero_hbm, acc)

        pltpu.sync_copy(didx_hbm.at[wid], idxv)
        pltpu.sync_copy(msg_hbm.at[pl.ds(wid * epw, epw)], msgv)
        plsc.subcore_barrier()

        def body(j, carry):
            pltpu.sync_copy(msgv.at[pl.ds(j * CHUNK, CHUNK)],
                            acc.at[idxv.at[j]], add=True)
            return carry

        lax.fori_loop(0, cpw, body, 0)
        plsc.subcore_barrier()

        @pl.when(sid == 0)
        def _():
            pltpu.sync_copy(acc, out_hbm.at[cid])

    return k


def _dense_body(et_ref, x_ref, y_ref, w1_ref, w2_ref,
                lm_ref, pm_ref, bsel_ref, qx_ref, qy_ref, selk_ref,
                out_ref):
    et = et_ref[...]
    h = jax.nn.silu(
        jnp.dot(et, w1_ref[...], preferred_element_type=jnp.float32) * 0.25
    ) * ACT_CST

    bf = jnp.bfloat16
    w = jnp.dot(h.astype(bf), w2_ref[...],
                preferred_element_type=jnp.float32)
    lm = lm_ref[...]
    bsel = bsel_ref[...]
    tx = jnp.dot(lm, x_ref[...].astype(bf),
                 preferred_element_type=jnp.float32).astype(bf) * bsel
    ty = jnp.dot(lm, y_ref[...].astype(bf),
                 preferred_element_type=jnp.float32).astype(bf) * bsel
    xe = jnp.dot(tx, qx_ref[...], preferred_element_type=jnp.float32)
    ye = jnp.dot(ty, qy_ref[...], preferred_element_type=jnp.float32)
    prod = w * xe * ye
    m128 = jnp.dot(prod.astype(bf), selk_ref[...],
                   preferred_element_type=jnp.float32)
    out_ref[...] = jnp.dot(pm_ref[...], (m128 * bsel).astype(bf),
                           preferred_element_type=jnp.float32)


def _dense_consts(B):
    scale = 1.0 / (8.0 * MUL * float(NUM_NEIGHBORS) ** 0.5)
    e = jnp.arange(B, dtype=jnp.int32)
    r = jnp.arange(B // 16, dtype=jnp.int32)
    l = jnp.arange(128, dtype=jnp.int32)
    m = jnp.arange(WNUM, dtype=jnp.int32)
    bf = jnp.bfloat16
    lm = (e[:, None] // 16 == r[None, :]).astype(bf)
    pm = (r[:, None] == e[None, :] // 16).astype(bf)
    bsel = (l[None, :] // MUL == e[:, None] % 16).astype(bf)
    qx = (l[:, None] % MUL == m[None, :] // (MUL * MUL)).astype(bf)
    qy = (l[:, None] % MUL == (m[None, :] // MUL) % MUL).astype(bf)
    selk = (m[:, None] % MUL == l[None, :] % MUL).astype(bf) * bf(scale)
    return lm, pm, bsel, qx, qy, selk


def _dense_fn(E_pad, B):
    grid = E_pad // B
    R = B * MUL // 128
    return pl.pallas_call(
        _dense_body,
        grid=(grid,),
        in_specs=[
            pl.BlockSpec((B, FC_IN), lambda i: (i, 0)),
            pl.BlockSpec((R, 128), lambda i: (i, 0)),
            pl.BlockSpec((R, 128), lambda i: (i, 0)),
            pl.BlockSpec((FC_IN, FC_HID), lambda i: (0, 0)),
            pl.BlockSpec((FC_HID, WNUM), lambda i: (0, 0)),
            pl.BlockSpec((B, R), lambda i: (0, 0)),
            pl.BlockSpec((R, B), lambda i: (0, 0)),
            pl.BlockSpec((B, 128), lambda i: (0, 0)),
            pl.BlockSpec((128, WNUM), lambda i: (0, 0)),
            pl.BlockSpec((128, WNUM), lambda i: (0, 0)),
            pl.BlockSpec((WNUM, 128), lambda i: (0, 0)),
        ],
        out_specs=pl.BlockSpec((R, 128), lambda i: (i, 0)),
        out_shape=jax.ShapeDtypeStruct((E_pad * MUL // 128, 128), jnp.float32),
        compiler_params=pltpu.CompilerParams(
            dimension_semantics=("parallel",)),
    )


def _combine_body(p_ref, o_ref):
    o_ref[...] = p_ref[0] + p_ref[1]


def _combine_fn(N):
    return pl.pallas_call(
        _combine_body,
        out_shape=jax.ShapeDtypeStruct((N, MUL), jnp.float32),
    )


def kernel(edge_src, edge_dst, node_emb, edge_type, W1, W2):
    E = edge_src.shape[0]
    N = node_emb.shape[1]
    quantum = NW * CHUNK
    E_pad = ((E + quantum - 1) // quantum) * quantum
    cpw = E_pad // quantum
    pad = E_pad - E

    pad_idx = jnp.arange(pad, dtype=jnp.int32) % N
    src_p = jnp.concatenate([edge_src, pad_idx])
    dst_p = jnp.concatenate([edge_dst, pad_idx])
    et_p = jnp.concatenate(
        [edge_type, jnp.zeros((pad, edge_type.shape[1]), edge_type.dtype)])
    emb = node_emb[0]

    src_emb, dst_emb = _gather_fn(E_pad, N)(src_p, dst_p, emb)
    xp = src_emb.reshape(E_pad * MUL // 128, 128)
    yp = dst_emb.reshape(E_pad * MUL // 128, 128)

    msgs = _dense_fn(E_pad, 2048)(et_p, xp, yp, W1,
                                  W2.astype(jnp.bfloat16), *_dense_consts(2048))

    partials = _scatter_fn(N, cpw)(
        dst_p.reshape(NW, cpw, CHUNK), msgs.reshape(E_pad, MUL),
        jnp.zeros((N, MUL), jnp.float32))

    out = _combine_fn(N)(partials)
    return out.reshape(node_emb.shape)


# speedup vs baseline: 1.0562x; 1.0302x over previous
import functools

import jax
import jax.numpy as jnp
from jax import lax
from jax.experimental import pallas as pl
from jax.experimental.pallas import tpu as pltpu
from jax.experimental.pallas import tpu_sc as plsc

MUL = 8
FC_IN = 16
FC_HID = 64
WNUM = MUL * MUL * MUL
ACT_CST = 1.679
NUM_NEIGHBORS = 16.0

NC, NS = 2, 16
NW = NC * NS
CHUNK = 128

_MESH = dict(core_axis_name="c", subcore_axis_name="s")


@functools.lru_cache(maxsize=None)
def _gather_fn(E_pad, N):
    mesh = plsc.VectorSubcoreMesh(**_MESH)
    epw = E_pad // NW

    @functools.partial(
        pl.kernel,
        out_type=(
            jax.ShapeDtypeStruct((E_pad, MUL), jnp.float32),
            jax.ShapeDtypeStruct((E_pad, MUL), jnp.float32),
        ),
        mesh=mesh,
        scratch_types=[
            pltpu.VMEM((epw,), jnp.int32),
            pltpu.VMEM((epw,), jnp.int32),
            pltpu.VMEM((epw, MUL), jnp.float32),
            pltpu.VMEM((epw, MUL), jnp.float32),
            pltpu.VMEM_SHARED((N, MUL), jnp.float32),
            pltpu.SemaphoreType.DMA,
        ],
        compiler_params=pltpu.CompilerParams(use_tc_tiling_on_sc=False),
    )
    def k(src_hbm, dst_hbm, emb_hbm, src_out, dst_out,
          sidx, didx, srows, drows, emb_sh, sem):
        sid = lax.axis_index("s")
        wid = sid * NC + lax.axis_index("c")
        base = wid * epw
        rps = N // NS
        pltpu.sync_copy(emb_hbm.at[pl.ds(sid * rps, rps)],
                        emb_sh.at[pl.ds(sid * rps, rps)])
        pltpu.sync_copy(src_hbm.at[pl.ds(base, epw)], sidx)
        pltpu.sync_copy(dst_hbm.at[pl.ds(base, epw)], didx)
        plsc.subcore_barrier()
        c1 = pltpu.async_copy(emb_sh.at[sidx], srows, sem)
        c2 = pltpu.async_copy(emb_sh.at[didx], drows, sem)
        c1.wait()
        c2.wait()
        pltpu.sync_copy(srows, src_out.at[pl.ds(base, epw)])
        pltpu.sync_copy(drows, dst_out.at[pl.ds(base, epw)])

    return k


@functools.lru_cache(maxsize=None)
def _scatter_fn(N, cpw):
    mesh = plsc.VectorSubcoreMesh(**_MESH)
    epw = cpw * CHUNK

    @functools.partial(
        pl.kernel,
        out_type=jax.ShapeDtypeStruct((NC, N, MUL), jnp.float32),
        mesh=mesh,
        scratch_types=[
            pltpu.VMEM((cpw, CHUNK), jnp.int32),
            pltpu.VMEM((epw, MUL), jnp.float32),
            pltpu.VMEM_SHARED((N, MUL), jnp.float32),
            pltpu.SemaphoreType.DMA,
        ],
        compiler_params=pltpu.CompilerParams(use_tc_tiling_on_sc=False),
    )
    def k(didx_hbm, msg_hbm, zero_hbm, out_hbm, idxv, msgv, acc, sem):
        cid = lax.axis_index("c")
        sid = lax.axis_index("s")
        wid = sid * NC + cid

        @pl.when(sid == 0)
        def _():
            pltpu.sync_copy(zero_hbm, acc)

        pltpu.sync_copy(didx_hbm.at[wid], idxv)
        pltpu.sync_copy(msg_hbm.at[pl.ds(wid * epw, epw)], msgv)
        plsc.subcore_barrier()

        def body(j, carry):
            pltpu.sync_copy(msgv.at[pl.ds(j * CHUNK, CHUNK)],
                            acc.at[idxv.at[j]], add=True)
            return carry

        lax.fori_loop(0, cpw, body, 0)
        plsc.subcore_barrier()

        @pl.when(sid == 0)
        def _():
            pltpu.sync_copy(acc, out_hbm.at[cid])

    return k


def _dense_body(et_ref, x_ref, y_ref, w1_ref, w2_ref,
                lm_ref, pm_ref, bsel_ref, qx_ref, qy_ref, selk_ref,
                out_ref):
    et = et_ref[...]
    h = jax.nn.silu(
        jnp.dot(et, w1_ref[...], preferred_element_type=jnp.float32) * 0.25
    ) * ACT_CST

    bf = jnp.bfloat16
    w = jnp.dot(h.astype(bf), w2_ref[...],
                preferred_element_type=jnp.float32)
    lm = lm_ref[...]
    bsel = bsel_ref[...]
    tx = jnp.dot(lm, x_ref[...].astype(bf),
                 preferred_element_type=jnp.float32).astype(bf) * bsel
    ty = jnp.dot(lm, y_ref[...].astype(bf),
                 preferred_element_type=jnp.float32).astype(bf) * bsel
    xe = jnp.dot(tx, qx_ref[...], preferred_element_type=jnp.float32)
    ye = jnp.dot(ty, qy_ref[...], preferred_element_type=jnp.float32)
    prod = w * xe * ye
    m128 = jnp.dot(prod.astype(bf), selk_ref[...],
                   preferred_element_type=jnp.float32)
    out_ref[...] = jnp.dot(pm_ref[...], (m128 * bsel).astype(bf),
                           preferred_element_type=jnp.float32)


def _dense_consts(B):
    scale = 1.0 / (8.0 * MUL * float(NUM_NEIGHBORS) ** 0.5)
    e = jnp.arange(B, dtype=jnp.int32)
    r = jnp.arange(B // 16, dtype=jnp.int32)
    l = jnp.arange(128, dtype=jnp.int32)
    m = jnp.arange(WNUM, dtype=jnp.int32)
    bf = jnp.bfloat16
    lm = (e[:, None] // 16 == r[None, :]).astype(bf)
    pm = (r[:, None] == e[None, :] // 16).astype(bf)
    bsel = (l[None, :] // MUL == e[:, None] % 16).astype(bf)
    qx = (l[:, None] % MUL == m[None, :] // (MUL * MUL)).astype(bf)
    qy = (l[:, None] % MUL == (m[None, :] // MUL) % MUL).astype(bf)
    selk = (m[:, None] % MUL == l[None, :] % MUL).astype(bf) * bf(scale)
    return lm, pm, bsel, qx, qy, selk


def _dense_fn(E_pad, B):
    grid = E_pad // B
    R = B * MUL // 128
    return pl.pallas_call(
        _dense_body,
        grid=(grid,),
        in_specs=[
            pl.BlockSpec((B, FC_IN), lambda i: (i, 0)),
            pl.BlockSpec((R, 128), lambda i: (i, 0)),
            pl.BlockSpec((R, 128), lambda i: (i, 0)),
            pl.BlockSpec((FC_IN, FC_HID), lambda i: (0, 0)),
            pl.BlockSpec((FC_HID, WNUM), lambda i: (0, 0)),
            pl.BlockSpec((B, R), lambda i: (0, 0)),
            pl.BlockSpec((R, B), lambda i: (0, 0)),
            pl.BlockSpec((B, 128), lambda i: (0, 0)),
            pl.BlockSpec((128, WNUM), lambda i: (0, 0)),
            pl.BlockSpec((128, WNUM), lambda i: (0, 0)),
            pl.BlockSpec((WNUM, 128), lambda i: (0, 0)),
        ],
        out_specs=pl.BlockSpec((R, 128), lambda i: (i, 0)),
        out_shape=jax.ShapeDtypeStruct((E_pad * MUL // 128, 128), jnp.float32),
        compiler_params=pltpu.CompilerParams(
            dimension_semantics=("parallel",)),
    )


def _combine_body(p_ref, o_ref):
    o_ref[...] = p_ref[0] + p_ref[1]


def _combine_fn(N):
    return pl.pallas_call(
        _combine_body,
        out_shape=jax.ShapeDtypeStruct((N, MUL), jnp.float32),
    )


def kernel(edge_src, edge_dst, node_emb, edge_type, W1, W2):
    E = edge_src.shape[0]
    N = node_emb.shape[1]
    quantum = NW * CHUNK
    E_pad = ((E + quantum - 1) // quantum) * quantum
    cpw = E_pad // quantum
    pad = E_pad - E

    pad_idx = jnp.arange(pad, dtype=jnp.int32) % N
    src_p = jnp.concatenate([edge_src, pad_idx])
    dst_p = jnp.concatenate([edge_dst, pad_idx])
    et_p = jnp.concatenate(
        [edge_type, jnp.zeros((pad, edge_type.shape[1]), edge_type.dtype)])
    emb = node_emb[0]

    src_emb, dst_emb = _gather_fn(E_pad, N)(src_p, dst_p, emb)
    xp = src_emb.reshape(E_pad * MUL // 128, 128)
    yp = dst_emb.reshape(E_pad * MUL // 128, 128)

    msgs = _dense_fn(E_pad, 2048)(et_p, xp, yp, W1,
                                  W2.astype(jnp.bfloat16), *_dense_consts(2048))

    partials = _scatter_fn(N, cpw)(
        dst_p.reshape(NW, cpw, CHUNK), msgs.reshape(E_pad, MUL),
        jnp.zeros((N, MUL), jnp.float32))

    out = _combine_fn(N)(partials)
    return out.reshape(node_emb.shape)


# speedup vs baseline: 1.0570x; 1.0008x over previous
import functools

import jax
import jax.numpy as jnp
from jax import lax
from jax.experimental import pallas as pl
from jax.experimental.pallas import tpu as pltpu
from jax.experimental.pallas import tpu_sc as plsc

MUL = 8
FC_IN = 16
FC_HID = 64
WNUM = MUL * MUL * MUL
ACT_CST = 1.679
NUM_NEIGHBORS = 16.0

NC, NS = 2, 16
NW = NC * NS
CHUNK = 128

_MESH = dict(core_axis_name="c", subcore_axis_name="s")


@functools.lru_cache(maxsize=None)
def _gather_fn(E_pad, N):
    mesh = plsc.VectorSubcoreMesh(**_MESH)
    epw = E_pad // NW

    @functools.partial(
        pl.kernel,
        out_type=(
            jax.ShapeDtypeStruct((E_pad, MUL), jnp.float32),
            jax.ShapeDtypeStruct((E_pad, MUL), jnp.float32),
        ),
        mesh=mesh,
        scratch_types=[
            pltpu.VMEM((epw,), jnp.int32),
            pltpu.VMEM((epw,), jnp.int32),
            pltpu.VMEM((epw, MUL), jnp.float32),
            pltpu.VMEM((epw, MUL), jnp.float32),
            pltpu.VMEM_SHARED((N, MUL), jnp.float32),
            pltpu.SemaphoreType.DMA,
        ],
        compiler_params=pltpu.CompilerParams(use_tc_tiling_on_sc=False),
    )
    def k(src_hbm, dst_hbm, emb_hbm, src_out, dst_out,
          sidx, didx, srows, drows, emb_sh, sem):
        sid = lax.axis_index("s")
        wid = sid * NC + lax.axis_index("c")
        base = wid * epw
        rps = N // NS
        pltpu.sync_copy(emb_hbm.at[pl.ds(sid * rps, rps)],
                        emb_sh.at[pl.ds(sid * rps, rps)])
        pltpu.sync_copy(src_hbm.at[pl.ds(base, epw)], sidx)
        pltpu.sync_copy(dst_hbm.at[pl.ds(base, epw)], didx)
        plsc.subcore_barrier()
        c1 = pltpu.async_copy(emb_sh.at[sidx], srows, sem)
        c2 = pltpu.async_copy(emb_sh.at[didx], drows, sem)
        c1.wait()
        c2.wait()
        pltpu.sync_copy(srows, src_out.at[pl.ds(base, epw)])
        pltpu.sync_copy(drows, dst_out.at[pl.ds(base, epw)])

    return k


@functools.lru_cache(maxsize=None)
def _scatter_fn(N, cpw):
    mesh = plsc.VectorSubcoreMesh(**_MESH)
    epw = cpw * CHUNK

    @functools.partial(
        pl.kernel,
        out_type=jax.ShapeDtypeStruct((NC, N, MUL), jnp.float32),
        mesh=mesh,
        scratch_types=[
            pltpu.VMEM((cpw, CHUNK), jnp.int32),
            pltpu.VMEM((epw, MUL), jnp.float32),
            pltpu.VMEM_SHARED((N, MUL), jnp.float32),
            pltpu.SemaphoreType.DMA,
        ],
        compiler_params=pltpu.CompilerParams(use_tc_tiling_on_sc=False),
    )
    def k(didx_hbm, msg_hbm, zero_hbm, out_hbm, idxv, msgv, acc, sem):
        cid = lax.axis_index("c")
        sid = lax.axis_index("s")
        wid = sid * NC + cid

        rps = N // NS
        pltpu.sync_copy(zero_hbm.at[pl.ds(sid * rps, rps)],
                        acc.at[pl.ds(sid * rps, rps)])
        pltpu.sync_copy(didx_hbm.at[wid], idxv)
        pltpu.sync_copy(msg_hbm.at[pl.ds(wid * epw, epw)], msgv)
        plsc.subcore_barrier()

        def body(j, carry):
            pltpu.sync_copy(msgv.at[pl.ds(j * CHUNK, CHUNK)],
                            acc.at[idxv.at[j]], add=True)
            return carry

        lax.fori_loop(0, cpw, body, 0)
        plsc.subcore_barrier()

        pltpu.sync_copy(acc.at[pl.ds(sid * rps, rps)],
                        out_hbm.at[cid].at[pl.ds(sid * rps, rps)])

    return k


def _dense_body(et_ref, x_ref, y_ref, w1_ref, w2_ref,
                lm_ref, pm_ref, bsel_ref, qx_ref, qy_ref, selk_ref,
                out_ref):
    et = et_ref[...]
    h = jax.nn.silu(
        jnp.dot(et, w1_ref[...], preferred_element_type=jnp.float32) * 0.25
    ) * ACT_CST

    bf = jnp.bfloat16
    w = jnp.dot(h.astype(bf), w2_ref[...],
                preferred_element_type=jnp.float32)
    lm = lm_ref[...]
    bsel = bsel_ref[...]
    tx = jnp.dot(lm, x_ref[...].astype(bf),
                 preferred_element_type=jnp.float32).astype(bf) * bsel
    ty = jnp.dot(lm, y_ref[...].astype(bf),
                 preferred_element_type=jnp.float32).astype(bf) * bsel
    xe = jnp.dot(tx, qx_ref[...], preferred_element_type=jnp.float32)
    ye = jnp.dot(ty, qy_ref[...], preferred_element_type=jnp.float32)
    prod = w * xe * ye
    m128 = jnp.dot(prod.astype(bf), selk_ref[...],
                   preferred_element_type=jnp.float32)
    out_ref[...] = jnp.dot(pm_ref[...], (m128 * bsel).astype(bf),
                           preferred_element_type=jnp.float32)


def _dense_consts(B):
    scale = 1.0 / (8.0 * MUL * float(NUM_NEIGHBORS) ** 0.5)
    e = jnp.arange(B, dtype=jnp.int32)
    r = jnp.arange(B // 16, dtype=jnp.int32)
    l = jnp.arange(128, dtype=jnp.int32)
    m = jnp.arange(WNUM, dtype=jnp.int32)
    bf = jnp.bfloat16
    lm = (e[:, None] // 16 == r[None, :]).astype(bf)
    pm = (r[:, None] == e[None, :] // 16).astype(bf)
    bsel = (l[None, :] // MUL == e[:, None] % 16).astype(bf)
    qx = (l[:, None] % MUL == m[None, :] // (MUL * MUL)).astype(bf)
    qy = (l[:, None] % MUL == (m[None, :] // MUL) % MUL).astype(bf)
    selk = (m[:, None] % MUL == l[None, :] % MUL).astype(bf) * bf(scale)
    return lm, pm, bsel, qx, qy, selk


def _dense_fn(E_pad, B):
    grid = E_pad // B
    R = B * MUL // 128
    return pl.pallas_call(
        _dense_body,
        grid=(grid,),
        in_specs=[
            pl.BlockSpec((B, FC_IN), lambda i: (i, 0)),
            pl.BlockSpec((R, 128), lambda i: (i, 0)),
            pl.BlockSpec((R, 128), lambda i: (i, 0)),
            pl.BlockSpec((FC_IN, FC_HID), lambda i: (0, 0)),
            pl.BlockSpec((FC_HID, WNUM), lambda i: (0, 0)),
            pl.BlockSpec((B, R), lambda i: (0, 0)),
            pl.BlockSpec((R, B), lambda i: (0, 0)),
            pl.BlockSpec((B, 128), lambda i: (0, 0)),
            pl.BlockSpec((128, WNUM), lambda i: (0, 0)),
            pl.BlockSpec((128, WNUM), lambda i: (0, 0)),
            pl.BlockSpec((WNUM, 128), lambda i: (0, 0)),
        ],
        out_specs=pl.BlockSpec((R, 128), lambda i: (i, 0)),
        out_shape=jax.ShapeDtypeStruct((E_pad * MUL // 128, 128), jnp.float32),
        compiler_params=pltpu.CompilerParams(
            dimension_semantics=("parallel",)),
    )


def _combine_body(p_ref, o_ref):
    o_ref[...] = p_ref[0] + p_ref[1]


def _combine_fn(N):
    return pl.pallas_call(
        _combine_body,
        out_shape=jax.ShapeDtypeStruct((N, MUL), jnp.float32),
    )


def kernel(edge_src, edge_dst, node_emb, edge_type, W1, W2):
    E = edge_src.shape[0]
    N = node_emb.shape[1]
    quantum = NW * CHUNK
    E_pad = ((E + quantum - 1) // quantum) * quantum
    cpw = E_pad // quantum
    pad = E_pad - E

    pad_idx = jnp.arange(pad, dtype=jnp.int32) % N
    src_p = jnp.concatenate([edge_src, pad_idx])
    dst_p = jnp.concatenate([edge_dst, pad_idx])
    et_p = jnp.concatenate(
        [edge_type, jnp.zeros((pad, edge_type.shape[1]), edge_type.dtype)])
    emb = node_emb[0]

    src_emb, dst_emb = _gather_fn(E_pad, N)(src_p, dst_p, emb)
    xp = src_emb.reshape(E_pad * MUL // 128, 128)
    yp = dst_emb.reshape(E_pad * MUL // 128, 128)

    msgs = _dense_fn(E_pad, 2048)(et_p, xp, yp, W1,
                                  W2.astype(jnp.bfloat16), *_dense_consts(2048))

    partials = _scatter_fn(N, cpw)(
        dst_p.reshape(NW, cpw, CHUNK), msgs.reshape(E_pad, MUL),
        jnp.zeros((N, MUL), jnp.float32))

    out = _combine_fn(N)(partials)
    return out.reshape(node_emb.shape)
